# baseline (speedup 1.0000x reference)
import functools

import jax
import jax.numpy as jnp
from jax import lax
from jax.experimental import pallas as pl
from jax.experimental.pallas import tpu as pltpu
from jax.experimental.pallas import tpu_sc as plsc

B = 16384
K = 8192
D = 256
COMMIT = 0.25

BM = 512
NB = B // BM

NC = 2
NS = 16
NW = NC * NS
ROWS_PER_W = B // NW
CHUNK = 256


def _dist_body(z_ref, cb_ref, idx_ref, loss_ref, acc_ref, esq_ref):
    i = pl.program_id(0)
    z = z_ref[...]

    @pl.when(i == 0)
    def _():
        cb = cb_ref[...]
        esq_ref[0, :] = jnp.sum(cb ** 2, axis=-1)
        acc_ref[0] = 0.0

    z_sq = jnp.sum(z ** 2, axis=-1, keepdims=True)
    e_sq = esq_ref[...]
    ze2 = lax.dot_general(z + z, cb_ref[...], (((1,), (1,)), ((), ())),
                          preferred_element_type=jnp.float32)
    distances = z_sq + e_sq - ze2
    mind = jnp.min(distances, axis=-1)
    cols = lax.broadcasted_iota(jnp.int32, (1, K), 1).astype(jnp.float32)
    hit = jnp.where(distances == mind[:, None], cols, float(K))
    idx_ref[0, 0, :] = jnp.min(hit, axis=-1).astype(jnp.int32)

    acc_ref[0] += jnp.sum(mind)

    @pl.when(i == NB - 1)
    def _():
        loss_ref[0, 0] = acc_ref[0] * ((1.0 + COMMIT) / (B * D))


_dist_call = pl.pallas_call(
    _dist_body,
    grid=(NB,),
    in_specs=[
        pl.BlockSpec((BM, D), lambda i: (i, 0)),
        pl.BlockSpec((K, D), lambda i: (0, 0)),
    ],
    out_specs=[
        pl.BlockSpec((1, 1, BM), lambda i: (i, 0, 0)),
        pl.BlockSpec(memory_space=pltpu.SMEM, block_shape=(1, 1),
                     index_map=lambda i: (0, 0)),
    ],
    out_shape=[
        jax.ShapeDtypeStruct((NB, 1, BM), jnp.int32),
        jax.ShapeDtypeStruct((1, 1), jnp.float32),
    ],
    scratch_shapes=[pltpu.SMEM((1,), jnp.float32),
                    pltpu.VMEM((1, K), jnp.float32)],
)


def _gather_body(cb_hbm, idx_hbm, out_hbm, idx_v, rows_v, sem):
    wid = lax.axis_index("s") * NC + lax.axis_index("c")
    base = wid * ROWS_PER_W
    for c in range(ROWS_PER_W // CHUNK):
        off = base + c * CHUNK
        pltpu.sync_copy(idx_hbm.at[pl.ds(off, CHUNK)], idx_v)
        pltpu.async_copy(cb_hbm.at[idx_v], rows_v, sem).wait()
        pltpu.sync_copy(rows_v, out_hbm.at[pl.ds(off, CHUNK)])


@functools.cache
def _gather_call():
    return functools.partial(
        pl.kernel,
        out_type=jax.ShapeDtypeStruct((B, D), jnp.float32),
        mesh=plsc.VectorSubcoreMesh(core_axis_name="c", subcore_axis_name="s"),
        scratch_types=[
            pltpu.VMEM((CHUNK,), jnp.int32),
            pltpu.VMEM((CHUNK, D), jnp.float32),
            pltpu.SemaphoreType.DMA,
        ],
    )(_gather_body)


def kernel(z, codebook):
    idx_blocks, loss = _dist_call(z, codebook)
    indices = idx_blocks.reshape(B)
    z_q_out = _gather_call()(codebook, indices)
    return (z_q_out, loss.reshape(()), indices)


# speedup vs baseline: 1.0756x; 1.0756x over previous
import functools

import jax
import jax.numpy as jnp
from jax import lax
from jax.experimental import pallas as pl
from jax.experimental.pallas import tpu as pltpu
from jax.experimental.pallas import tpu_sc as plsc

B = 16384
K = 8192
D = 256
COMMIT = 0.25

BM = 512
NB = B // BM

NC = 2
NS = 16
NW = NC * NS
ROWS_PER_W = B // NW
CHUNK = 256


def _dist_body(z_ref, cb_ref, idx_ref, loss_ref, acc_ref, esq_ref):
    i = pl.program_id(0)
    z = z_ref[...]

    @pl.when(i == 0)
    def _():
        cb = cb_ref[...]
        esq_ref[0, :] = jnp.sum(cb ** 2, axis=-1)
        acc_ref[0] = 0.0

    z_sq = jnp.sum(z ** 2, axis=-1, keepdims=True)
    e_sq = esq_ref[...]
    ze2 = lax.dot_general(z + z, cb_ref[...], (((1,), (1,)), ((), ())),
                          preferred_element_type=jnp.float32)
    distances = z_sq + e_sq - ze2
    mind = jnp.min(distances, axis=-1)
    idx_ref[0, 0, :] = jnp.argmin(distances, axis=-1).astype(jnp.int32)

    acc_ref[0] += jnp.sum(mind)

    @pl.when(i == NB - 1)
    def _():
        loss_ref[0, 0] = acc_ref[0] * ((1.0 + COMMIT) / (B * D))


_dist_call = pl.pallas_call(
    _dist_body,
    grid=(NB,),
    in_specs=[
        pl.BlockSpec((BM, D), lambda i: (i, 0)),
        pl.BlockSpec((K, D), lambda i: (0, 0)),
    ],
    out_specs=[
        pl.BlockSpec((1, 1, BM), lambda i: (i, 0, 0)),
        pl.BlockSpec(memory_space=pltpu.SMEM, block_shape=(1, 1),
                     index_map=lambda i: (0, 0)),
    ],
    out_shape=[
        jax.ShapeDtypeStruct((NB, 1, BM), jnp.int32),
        jax.ShapeDtypeStruct((1, 1), jnp.float32),
    ],
    scratch_shapes=[pltpu.SMEM((1,), jnp.float32),
                    pltpu.VMEM((1, K), jnp.float32)],
)


def _gather_body(cb_hbm, idx_hbm, out_hbm, idx_v, rows_v, sem):
    wid = lax.axis_index("s") * NC + lax.axis_index("c")
    base = wid * ROWS_PER_W
    for c in range(ROWS_PER_W // CHUNK):
        off = base + c * CHUNK
        pltpu.sync_copy(idx_hbm.at[pl.ds(off, CHUNK)], idx_v)
        pltpu.async_copy(cb_hbm.at[idx_v], rows_v, sem).wait()
        pltpu.sync_copy(rows_v, out_hbm.at[pl.ds(off, CHUNK)])


@functools.cache
def _gather_call():
    return functools.partial(
        pl.kernel,
        out_type=jax.ShapeDtypeStruct((B, D), jnp.float32),
        mesh=plsc.VectorSubcoreMesh(core_axis_name="c", subcore_axis_name="s"),
        scratch_types=[
            pltpu.VMEM((CHUNK,), jnp.int32),
            pltpu.VMEM((CHUNK, D), jnp.float32),
            pltpu.SemaphoreType.DMA,
        ],
    )(_gather_body)


def kernel(z, codebook):
    idx_blocks, loss = _dist_call(z, codebook)
    indices = idx_blocks.reshape(B)
    z_q_out = _gather_call()(codebook, indices)
    return (z_q_out, loss.reshape(()), indices)
